# baseline (speedup 1.0000x reference)
import functools

import numpy as np
import jax
import jax.numpy as jnp
from jax import lax
from jax.experimental import pallas as pl
from jax.experimental.pallas import tpu as pltpu
from jax.experimental.pallas import tpu_sc as plsc

_H = _W = 512
_HW = _H * _W
_B = 2
_G = int(_HW * 0.15)
_NSUB = 16
_CHUNK = 128
_CPW = 20
_GPW = _CHUNK * _CPW
_PPT = _HW // _NSUB
_SEG = 4096
_NSEG = _PPT // _SEG

_DCOS = np.float32(0.867)
_ETA = np.float32(1e-8)
_DXY = np.float32(0.005)
_DZ = np.float32(0.0001)


def _build_index_table():
    rng = np.random.RandomState(0)
    ps = []
    for _ in range(3):
        p = rng.choice(_HW, _G, replace=True)
        rng.shuffle(p)
        ps.append(p.astype(np.int32))
    npad = _NSUB * _GPW - _G
    pad = (np.arange(npad, dtype=np.int64) * 997 % _HW).astype(np.int32)
    r = np.zeros((3, _NSUB * _GPW), np.int32)
    for k in range(3):
        r[k, :_G] = ps[k]
        r[k, _G:] = pad
    return r.reshape(3, _NSUB, _CPW, _CHUNK).transpose(1, 2, 0, 3).copy()


_IDX = _build_index_table()


def _rsqrt_fast(x):
    i = lax.bitcast_convert_type(x, jnp.int32)
    y = lax.bitcast_convert_type(jnp.int32(0x5F3759DF) - (i >> 1), jnp.float32)
    for _ in range(2):
        y = y * (jnp.float32(1.5) - jnp.float32(0.5) * x * y * y)
    return y


def _cos_hit(e, eii, ejj):
    t = jnp.abs(e) - _DCOS * _ETA
    return (t > 0.0) & (t * t > (_DCOS * _DCOS) * (eii * ejj))


_mesh = plsc.VectorSubcoreMesh(core_axis_name="c", subcore_axis_name="s")


@functools.partial(
    pl.kernel,
    mesh=_mesh,
    compiler_params=pltpu.CompilerParams(
        needs_layout_passes=False, use_tc_tiling_on_sc=False
    ),
    out_type=jax.ShapeDtypeStruct((_B, _NSUB, 2, 16), jnp.float32),
    scratch_types=[
        pltpu.HBM((_B, _HW, 6), jnp.float32),
        pltpu.VMEM((_CPW, 3, _CHUNK), jnp.int32),
        pltpu.VMEM((2, 6, 4, 8, 128), jnp.float32),
        pltpu.VMEM((2, _SEG // 2, 6), jnp.float32),
        pltpu.VMEM((6, _CHUNK, 6), jnp.float32),
        pltpu.VMEM((2, 16), jnp.float32),
        pltpu.SemaphoreType.DMA,
        pltpu.SemaphoreType.DMA,
        pltpu.SemaphoreType.DMA,
        pltpu.SemaphoreType.DMA,
    ],
)
def _vn_kernel(gt_hbm, pr_hbm, idx_hbm, out_hbm, tab_hbm,
               idx_v, in_v, st_v, rows_v, part_v, semp, semo, sem0, sem1):
    cid = lax.axis_index("c")
    sid = lax.axis_index("s")
    sems = (sem0, sem1)
    lane = jnp.arange(16, dtype=jnp.int32)

    pltpu.sync_copy(idx_hbm.at[sid], idx_v)
    part_v[0, :] = jnp.zeros((16,), jnp.float32)
    part_v[1, :] = jnp.zeros((16,), jnp.float32)

    pix0 = sid * _PPT

    def fire_in(s, b):
        tr = sid * _NSEG + s
        for ch in range(3):
            pltpu.async_copy(gt_hbm.at[cid, ch, tr], in_v.at[b, ch], semp)
            pltpu.async_copy(pr_hbm.at[cid, ch, tr], in_v.at[b, 3 + ch], semp)

    def drain_in(s, b):
        tr = sid * _NSEG + s
        for ch in range(3):
            pltpu.make_async_copy(gt_hbm.at[cid, ch, tr], in_v.at[b, ch],
                                  semp).wait()
            pltpu.make_async_copy(pr_hbm.at[cid, ch, tr], in_v.at[b, 3 + ch],
                                  semp).wait()

    _STG = _SEG // 2

    def fire_out(gh):
        pltpu.async_copy(st_v.at[gh % 2],
                         tab_hbm.at[cid, pl.ds(pix0 + gh * _STG, _STG), :],
                         semo)

    def drain_out(gh):
        pltpu.make_async_copy(st_v.at[gh % 2],
                              tab_hbm.at[cid, pl.ds(pix0 + gh * _STG, _STG), :],
                              semo).wait()

    if True:
        fire_in(0, 0)
        for s in range(_NSEG):
            bi = s % 2
            if s + 1 < _NSEG:
                fire_in(s + 1, 1 - bi)
            drain_in(s, bi)
            for h in range(2):
                gh = s * 2 + h
                bs = gh % 2
                if gh >= 2:
                    drain_out(gh - 2)

                bs_c = jnp.full((16,), bs, jnp.int32)
                ch_c = [jnp.full((16,), ch, jnp.int32) for ch in range(6)]

                def interleave(i, carry):
                    for u in range(4):
                        jj = i * 4 + u
                        ii = h * 128 + jj
                        row = jj * 16 + lane
                        rr = ii // 32
                        tc = (ii % 32) // 8
                        cc = (ii % 8) * 16
                        for ch in range(6):
                            v = in_v[bi, ch, tc, rr, pl.ds(cc, 16)]
                            plsc.store_scatter(st_v, [bs_c, row, ch_c[ch]], v)
                    return carry

                lax.fori_loop(0, _STG // 64, interleave, 0)
                fire_out(gh)
        drain_out(2 * _NSEG - 2)
        drain_out(2 * _NSEG - 1)

        plsc.subcore_barrier()

    tab = tab_hbm.at[cid]

    def fire(c, buf):
        for k in range(3):
            pltpu.async_copy(tab.at[idx_v.at[c, k]], rows_v.at[buf * 3 + k],
                             sems[buf])

    def drain(c, buf):
        for k in range(3):
            pltpu.make_async_copy(tab.at[idx_v.at[c, k]],
                                  rows_v.at[buf * 3 + k], sems[buf]).wait()

    def compute(c, buf):
        base = sid * _GPW + c * _CHUNK
        acc = jnp.zeros((16,), jnp.float32)
        cnt = jnp.zeros((16,), jnp.float32)
        one = jnp.ones((16,), jnp.float32)
        zero = jnp.zeros((16,), jnp.float32)
        for sub in range(8):
            ro = sub * 16 + lane
            valid = (base + sub * 16 + lane) < _G

            def col(k, ch):
                return plsc.load_gather(
                    rows_v,
                    [jnp.full((16,), buf * 3 + k, jnp.int32), ro,
                     jnp.full((16,), ch, jnp.int32)],
                )

            ax, ay, az = col(0, 0), col(0, 1), col(0, 2)
            bx, by, bz = col(1, 0), col(1, 1), col(1, 2)
            cx, cy, cz = col(2, 0), col(2, 1), col(2, 2)

            d12x, d12y, d12z = bx - ax, by - ay, bz - az
            d13x, d13y, d13z = cx - ax, cy - ay, cz - az
            d23x, d23y, d23z = cx - bx, cy - by, cz - bz

            mask_pad = (az > _DZ) & (bz > _DZ) & (cz > _DZ)
            mx = ((jnp.abs(d12x) < _DXY) | (jnp.abs(d13x) < _DXY)
                  | (jnp.abs(d23x) < _DXY))
            my = ((jnp.abs(d12y) < _DXY) | (jnp.abs(d13y) < _DXY)
                  | (jnp.abs(d23y) < _DXY))
            mz = ((jnp.abs(d12z) < _DXY) | (jnp.abs(d13z) < _DXY)
                  | (jnp.abs(d23z) < _DXY))

            e11 = d12x * d12x + d12y * d12y + d12z * d12z
            e22 = d13x * d13x + d13y * d13y + d13z * d13z
            e33 = d23x * d23x + d23y * d23y + d23z * d23z
            e12 = d12x * d13x + d12y * d13y + d12z * d13z
            e13 = d12x * d23x + d12y * d23y + d12z * d23z
            e23 = d13x * d23x + d13y * d23y + d13z * d23z

            def f(m):
                return jnp.where(m, one, zero)

            hits = (f(_cos_hit(e11, e11, e11)) + f(_cos_hit(e22, e22, e22))
                    + f(_cos_hit(e33, e33, e33))
                    + 2.0 * (f(_cos_hit(e12, e11, e22))
                             + f(_cos_hit(e13, e11, e33))
                             + f(_cos_hit(e23, e22, e33))))
            mask_cos = hits > 3.5
            mask = mask_pad & jnp.logical_not((mx & my & mz) | mask_cos)

            nx = d12y * d13z - d12z * d13y
            ny = d12z * d13x - d12x * d13z
            nz = d12x * d13y - d12y * d13x
            ssq = nx * nx + ny * ny + nz * nz
            rinv = jnp.where(ssq == 0.0, jnp.float32(100.0), _rsqrt_fast(ssq))
            gnx, gny, gnz = nx * rinv, ny * rinv, nz * rinv

            pax, pay, paz = col(0, 3), col(0, 4), col(0, 5)
            pbx, pby, pbz = col(1, 3), col(1, 4), col(1, 5)
            pcx, pcy, pcz = col(2, 3), col(2, 4), col(2, 5)
            c1 = paz == 0.0
            c2 = pbz == 0.0
            c3 = pcz == 0.0
            sub4 = jnp.float32(0.0001)
            pax = jnp.where(c1, sub4, pax)
            pbx = jnp.where(c1, sub4, pbx)
            pcx = jnp.where(c1, sub4, pcx)
            pay = jnp.where(c2, sub4, pay)
            pby = jnp.where(c2, sub4, pby)
            pcy = jnp.where(c2, sub4, pcy)
            paz = jnp.where(c3, sub4, paz)
            pbz = jnp.where(c3, sub4, pbz)
            pcz = jnp.where(c3, sub4, pcz)

            q12x, q12y, q12z = pbx - pax, pby - pay, pbz - paz
            q13x, q13y, q13z = pcx - pax, pcy - pay, pcz - paz
            ux = q12y * q13z - q12z * q13y
            uy = q12z * q13x - q12x * q13z
            uz = q12x * q13y - q12y * q13x
            usq = ux * ux + uy * uy + uz * uz
            urinv = jnp.where(usq == 0.0, jnp.float32(100.0),
                              _rsqrt_fast(usq))
            dnx, dny, dnz = ux * urinv, uy * urinv, uz * urinv

            loss = (jnp.abs(gnx - dnx) + jnp.abs(gny - dny)
                    + jnp.abs(gnz - dnz))
            acc = acc + jnp.where(valid & mask, loss, zero)
            cnt = cnt + jnp.where(valid & mask, one, zero)

        part_v[0, :] = part_v[0, :] + acc
        part_v[1, :] = part_v[1, :] + cnt

    _gather_loop(fire, drain, compute)
    pltpu.sync_copy(part_v, out_hbm.at[cid, sid])


def _gather_loop(fire, drain, compute):
    fire(0, 0)

    def body2(i, carry):
        c0 = 2 * i
        fire(c0 + 1, 1)
        drain(c0, 0)
        compute(c0, 0)

        @pl.when(c0 + 2 < _CPW)
        def _():
            fire(c0 + 2, 0)

        drain(c0 + 1, 1)
        compute(c0 + 1, 1)
        return carry

    lax.fori_loop(0, _CPW // 2, body2, 0)


def kernel(gt, pred):
    gt_t = gt.reshape(_B, 3, 64, 8, 4, 128).transpose(0, 1, 2, 4, 3, 5)
    pr_t = pred.reshape(_B, 3, 64, 8, 4, 128).transpose(0, 1, 2, 4, 3, 5)
    parts = _vn_kernel(gt_t, pr_t, jnp.asarray(_IDX))
    total = jnp.sum(parts[:, :, 0, :])
    count = jnp.sum(parts[:, :, 1, :])
    return total / count


# speedup vs baseline: 1.0518x; 1.0518x over previous
import functools

import numpy as np
import jax
import jax.numpy as jnp
from jax import lax
from jax.experimental import pallas as pl
from jax.experimental.pallas import tpu as pltpu
from jax.experimental.pallas import tpu_sc as plsc

_H = _W = 512
_HW = _H * _W
_B = 2
_G = int(_HW * 0.15)
_NSUB = 16
_CHUNK = 128
_CPW = 20
_GPW = _CHUNK * _CPW
_PPT = _HW // _NSUB
_SEG = 4096
_NSEG = _PPT // _SEG

_DCOS = np.float32(0.867)
_ETA = np.float32(1e-8)
_DXY = np.float32(0.005)
_DZ = np.float32(0.0001)


def _build_index_table():
    rng = np.random.RandomState(0)
    ps = []
    for _ in range(3):
        p = rng.choice(_HW, _G, replace=True)
        rng.shuffle(p)
        ps.append(p.astype(np.int32))
    npad = _NSUB * _GPW - _G
    pad = (np.arange(npad, dtype=np.int64) * 997 % _HW).astype(np.int32)
    r = np.zeros((3, _NSUB * _GPW), np.int32)
    for k in range(3):
        r[k, :_G] = ps[k]
        r[k, _G:] = pad
    return r.reshape(3, _NSUB, _CPW, _CHUNK).transpose(1, 2, 0, 3).copy()


_IDX = _build_index_table()


def _rsqrt_fast(x):
    i = lax.bitcast_convert_type(x, jnp.int32)
    y = lax.bitcast_convert_type(jnp.int32(0x5F3759DF) - (i >> 1), jnp.float32)
    for _ in range(2):
        y = y * (jnp.float32(1.5) - jnp.float32(0.5) * x * y * y)
    return y


def _cos_hit(e, eii, ejj):
    t = jnp.abs(e) - _DCOS * _ETA
    return (t > 0.0) & (t * t > (_DCOS * _DCOS) * (eii * ejj))


_mesh = plsc.VectorSubcoreMesh(core_axis_name="c", subcore_axis_name="s")


@functools.partial(
    pl.kernel,
    mesh=_mesh,
    compiler_params=pltpu.CompilerParams(
        needs_layout_passes=False, use_tc_tiling_on_sc=False
    ),
    out_type=[
        jax.ShapeDtypeStruct((_B, _NSUB, 2, 16), jnp.float32),
        jax.ShapeDtypeStruct((_B, _HW, 8), jnp.float32),
    ],
    scratch_types=[
        pltpu.VMEM((_CPW, 3, _CHUNK), jnp.int32),
        pltpu.VMEM((2, 6, 4, 8, 128), jnp.float32),
        pltpu.VMEM((2, _SEG // 2, 8), jnp.float32),
        pltpu.VMEM((6, _CHUNK, 8), jnp.float32),
        pltpu.VMEM((2, 16), jnp.float32),
        pltpu.SemaphoreType.DMA,
        pltpu.SemaphoreType.DMA,
        pltpu.SemaphoreType.DMA,
        pltpu.SemaphoreType.DMA,
    ],
)
def _vn_kernel(gt_hbm, pr_hbm, idx_hbm, out_hbm, tab_hbm,
               idx_v, in_v, st_v, rows_v, part_v, semp, semo, sem0, sem1):
    cid = lax.axis_index("c")
    sid = lax.axis_index("s")
    sems = (sem0, sem1)
    lane = jnp.arange(16, dtype=jnp.int32)

    pltpu.sync_copy(idx_hbm.at[sid], idx_v)
    part_v[0, :] = jnp.zeros((16,), jnp.float32)
    part_v[1, :] = jnp.zeros((16,), jnp.float32)

    pix0 = sid * _PPT

    def fire_in(s, b):
        tr = sid * _NSEG + s
        for ch in range(3):
            pltpu.async_copy(gt_hbm.at[cid, ch, tr], in_v.at[b, ch], semp)
            pltpu.async_copy(pr_hbm.at[cid, ch, tr], in_v.at[b, 3 + ch], semp)

    def drain_in(s, b):
        tr = sid * _NSEG + s
        for ch in range(3):
            pltpu.make_async_copy(gt_hbm.at[cid, ch, tr], in_v.at[b, ch],
                                  semp).wait()
            pltpu.make_async_copy(pr_hbm.at[cid, ch, tr], in_v.at[b, 3 + ch],
                                  semp).wait()

    _STG = _SEG // 2

    def fire_out(gh):
        pltpu.async_copy(st_v.at[gh % 2],
                         tab_hbm.at[cid, pl.ds(pix0 + gh * _STG, _STG), :],
                         semo)

    def drain_out(gh):
        pltpu.make_async_copy(st_v.at[gh % 2],
                              tab_hbm.at[cid, pl.ds(pix0 + gh * _STG, _STG), :],
                              semo).wait()

    with jax.named_scope("vn_pack"):
        fire_in(0, 0)
        for s in range(_NSEG):
            bi = s % 2
            if s + 1 < _NSEG:
                fire_in(s + 1, 1 - bi)
            drain_in(s, bi)
            for h in range(2):
                gh = s * 2 + h
                bs = gh % 2
                if gh >= 2:
                    drain_out(gh - 2)

                def interleave(i, carry):
                    for u in range(2):
                        jj = i * 2 + u
                        ii = h * 128 + jj
                        row = jj * 16 + lane
                        rr = ii // 32
                        tc = (ii % 32) // 8
                        cc = (ii % 8) * 16
                        for ch in range(6):
                            v = in_v[bi, ch, tc, rr, pl.ds(cc, 16)]
                            plsc.store_scatter(
                                st_v,
                                [jnp.full((16,), bs, jnp.int32), row,
                                 jnp.full((16,), ch, jnp.int32)], v)
                    return carry

                lax.fori_loop(0, _STG // 32, interleave, 0)
                fire_out(gh)
        drain_out(2 * _NSEG - 2)
        drain_out(2 * _NSEG - 1)

        plsc.subcore_barrier()

    tab = tab_hbm.at[cid]

    def fire(c, buf):
        for k in range(3):
            pltpu.async_copy(tab.at[idx_v.at[c, k]], rows_v.at[buf * 3 + k],
                             sems[buf])

    def drain(c, buf):
        for k in range(3):
            pltpu.make_async_copy(tab.at[idx_v.at[c, k]],
                                  rows_v.at[buf * 3 + k], sems[buf]).wait()

    def compute(c, buf):
        base = sid * _GPW + c * _CHUNK
        acc = jnp.zeros((16,), jnp.float32)
        cnt = jnp.zeros((16,), jnp.float32)
        one = jnp.ones((16,), jnp.float32)
        zero = jnp.zeros((16,), jnp.float32)
        for sub in range(8):
            ro = sub * 16 + lane
            valid = (base + sub * 16 + lane) < _G

            def col(k, ch):
                return plsc.load_gather(
                    rows_v,
                    [jnp.full((16,), buf * 3 + k, jnp.int32), ro,
                     jnp.full((16,), ch, jnp.int32)],
                )

            ax, ay, az = col(0, 0), col(0, 1), col(0, 2)
            bx, by, bz = col(1, 0), col(1, 1), col(1, 2)
            cx, cy, cz = col(2, 0), col(2, 1), col(2, 2)

            d12x, d12y, d12z = bx - ax, by - ay, bz - az
            d13x, d13y, d13z = cx - ax, cy - ay, cz - az
            d23x, d23y, d23z = cx - bx, cy - by, cz - bz

            mask_pad = (az > _DZ) & (bz > _DZ) & (cz > _DZ)
            mx = ((jnp.abs(d12x) < _DXY) | (jnp.abs(d13x) < _DXY)
                  | (jnp.abs(d23x) < _DXY))
            my = ((jnp.abs(d12y) < _DXY) | (jnp.abs(d13y) < _DXY)
                  | (jnp.abs(d23y) < _DXY))
            mz = ((jnp.abs(d12z) < _DXY) | (jnp.abs(d13z) < _DXY)
                  | (jnp.abs(d23z) < _DXY))

            e11 = d12x * d12x + d12y * d12y + d12z * d12z
            e22 = d13x * d13x + d13y * d13y + d13z * d13z
            e33 = d23x * d23x + d23y * d23y + d23z * d23z
            e12 = d12x * d13x + d12y * d13y + d12z * d13z
            e13 = d12x * d23x + d12y * d23y + d12z * d23z
            e23 = d13x * d23x + d13y * d23y + d13z * d23z

            def f(m):
                return jnp.where(m, one, zero)

            hits = (f(_cos_hit(e11, e11, e11)) + f(_cos_hit(e22, e22, e22))
                    + f(_cos_hit(e33, e33, e33))
                    + 2.0 * (f(_cos_hit(e12, e11, e22))
                             + f(_cos_hit(e13, e11, e33))
                             + f(_cos_hit(e23, e22, e33))))
            mask_cos = hits > 3.5
            mask = mask_pad & jnp.logical_not((mx & my & mz) | mask_cos)

            nx = d12y * d13z - d12z * d13y
            ny = d12z * d13x - d12x * d13z
            nz = d12x * d13y - d12y * d13x
            ssq = nx * nx + ny * ny + nz * nz
            rinv = jnp.where(ssq == 0.0, jnp.float32(100.0), _rsqrt_fast(ssq))
            gnx, gny, gnz = nx * rinv, ny * rinv, nz * rinv

            pax, pay, paz = col(0, 3), col(0, 4), col(0, 5)
            pbx, pby, pbz = col(1, 3), col(1, 4), col(1, 5)
            pcx, pcy, pcz = col(2, 3), col(2, 4), col(2, 5)
            c1 = paz == 0.0
            c2 = pbz == 0.0
            c3 = pcz == 0.0
            sub4 = jnp.float32(0.0001)
            pax = jnp.where(c1, sub4, pax)
            pbx = jnp.where(c1, sub4, pbx)
            pcx = jnp.where(c1, sub4, pcx)
            pay = jnp.where(c2, sub4, pay)
            pby = jnp.where(c2, sub4, pby)
            pcy = jnp.where(c2, sub4, pcy)
            paz = jnp.where(c3, sub4, paz)
            pbz = jnp.where(c3, sub4, pbz)
            pcz = jnp.where(c3, sub4, pcz)

            q12x, q12y, q12z = pbx - pax, pby - pay, pbz - paz
            q13x, q13y, q13z = pcx - pax, pcy - pay, pcz - paz
            ux = q12y * q13z - q12z * q13y
            uy = q12z * q13x - q12x * q13z
            uz = q12x * q13y - q12y * q13x
            usq = ux * ux + uy * uy + uz * uz
            urinv = jnp.where(usq == 0.0, jnp.float32(100.0),
                              _rsqrt_fast(usq))
            dnx, dny, dnz = ux * urinv, uy * urinv, uz * urinv

            loss = (jnp.abs(gnx - dnx) + jnp.abs(gny - dny)
                    + jnp.abs(gnz - dnz))
            acc = acc + jnp.where(valid & mask, loss, zero)
            cnt = cnt + jnp.where(valid & mask, one, zero)

        part_v[0, :] = part_v[0, :] + acc
        part_v[1, :] = part_v[1, :] + cnt

    with jax.named_scope("vn_gather"):
        _gather_loop(fire, drain, compute)
    pltpu.sync_copy(part_v, out_hbm.at[cid, sid])


def _gather_loop(fire, drain, compute):
    fire(0, 0)

    def body2(i, carry):
        c0 = 2 * i
        fire(c0 + 1, 1)
        drain(c0, 0)
        compute(c0, 0)

        @pl.when(c0 + 2 < _CPW)
        def _():
            fire(c0 + 2, 0)

        drain(c0 + 1, 1)
        compute(c0 + 1, 1)
        return carry

    lax.fori_loop(0, _CPW // 2, body2, 0)


def kernel(gt, pred):
    gt_t = gt.reshape(_B, 3, 64, 8, 4, 128).transpose(0, 1, 2, 4, 3, 5)
    pr_t = pred.reshape(_B, 3, 64, 8, 4, 128).transpose(0, 1, 2, 4, 3, 5)
    parts, _ = _vn_kernel(gt_t, pr_t, jnp.asarray(_IDX))
    total = jnp.sum(parts[:, :, 0, :])
    count = jnp.sum(parts[:, :, 1, :])
    return total / count
